# baseline (speedup 1.0000x reference)
import functools

import jax
import jax.numpy as jnp
from jax import lax
from jax.experimental import pallas as pl
from jax.experimental.pallas import tpu as pltpu
from jax.experimental.pallas import tpu_sc as plsc

_D = 512
_NA = 8
_CH = 64


def _lookup(ids, atom_embeddings, lw_flat):
    B = ids.shape[0]
    info = plsc.get_sparse_core_info()
    nc, ns = info.num_cores, info.num_subcores
    nw = nc * ns
    b_per_w = B // nw
    n_ch = b_per_w // _CH

    mesh = plsc.VectorSubcoreMesh(core_axis_name="c", subcore_axis_name="s")

    @functools.partial(
        pl.kernel,
        mesh=mesh,
        out_type=(
            jax.ShapeDtypeStruct((B, _D), jnp.float32),
            jax.ShapeDtypeStruct((B * _NA,), jnp.float32),
        ),
        scratch_types=[
            pltpu.VMEM((b_per_w,), jnp.int32),
            pltpu.VMEM((b_per_w * _NA,), jnp.float32),
            pltpu.VMEM((_CH, _D), jnp.float32),
            pltpu.VMEM((_CH, _D), jnp.float32),
            pltpu.SemaphoreType.DMA,
            pltpu.SemaphoreType.DMA,
            pltpu.SemaphoreType.DMA,
        ],
    )
    def k(ids_hbm, atoms_hbm, lw_hbm, out_a, out_w,
          idx_v, wrows_v, buf0, buf1, sem0, sem1, semw):
        wid = lax.axis_index("s") * nc + lax.axis_index("c")
        base = wid * b_per_w

        pltpu.sync_copy(ids_hbm.at[pl.ds(base, b_per_w)], idx_v)

        bufs = (buf0, buf1)
        sems = (sem0, sem1)

        cp = pltpu.async_copy(
            atoms_hbm.at[idx_v.at[pl.ds(0, _CH)]], bufs[0], sems[0])
        prev = (cp, 0, 0)

        def wbody(g, carry):
            vec = idx_v[pl.ds(g * 16, 16)]
            for j in range(16):
                t = vec[j]
                pltpu.async_copy(lw_hbm.at[pl.ds(t * _NA, _NA)],
                                 wrows_v.at[pl.ds((g * 16 + j) * _NA, _NA)],
                                 semw)
            return carry
        lax.fori_loop(0, b_per_w // 16, wbody, 0)

        for c in range(1, n_ch):
            s = c & 1
            cp = pltpu.async_copy(
                atoms_hbm.at[idx_v.at[pl.ds(c * _CH, _CH)]], bufs[s], sems[s])
            pcp, ps, pc_i = prev
            pcp.wait()
            pltpu.sync_copy(bufs[ps],
                            out_a.at[pl.ds(base + pc_i * _CH, _CH)])
            prev = (cp, s, c)
        pcp, ps, pc_i = prev
        pcp.wait()
        pltpu.sync_copy(bufs[ps], out_a.at[pl.ds(base + pc_i * _CH, _CH)])

        pltpu.make_async_copy(lw_hbm.at[pl.ds(0, b_per_w * _NA)], wrows_v,
                              semw).wait()
        pltpu.sync_copy(wrows_v, out_w.at[pl.ds(base * _NA, b_per_w * _NA)])

    return k(ids, atom_embeddings, lw_flat)


def kernel(token_ids, atom_embeddings, log_weight_embeddings):
    Bt, S = token_ids.shape
    ids = token_ids.reshape(-1).astype(jnp.int32)
    lw_flat = log_weight_embeddings.reshape(-1)
    atoms_flat, lw = _lookup(ids, atom_embeddings, lw_flat)
    atoms = atoms_flat.reshape(Bt, S, _NA, _D // _NA)
    log_weights = lw.reshape(Bt, S, _NA)
    return (atoms, log_weights)


# speedup vs baseline: 1.7813x; 1.7813x over previous
import functools

import jax
import jax.numpy as jnp
from jax import lax
from jax.experimental import pallas as pl
from jax.experimental.pallas import tpu as pltpu
from jax.experimental.pallas import tpu_sc as plsc

_D = 512
_NA = 8
_DB = 64
_CH = 64
_TBLK = 1024


def _sc_lookup(ids, atom_embeddings, log_weight_embeddings):
    B = ids.shape[0]
    info = plsc.get_sparse_core_info()
    nc, ns = info.num_cores, info.num_subcores
    nw = nc * ns
    b_per_w = B // nw
    n_ch = b_per_w // _CH

    mesh = plsc.VectorSubcoreMesh(core_axis_name="c", subcore_axis_name="s")

    @functools.partial(
        pl.kernel,
        mesh=mesh,
        out_type=(
            jax.ShapeDtypeStruct((B, _D), jnp.float32),
            jax.ShapeDtypeStruct((B, _NA), jnp.float32),
        ),
        scratch_types=[
            pltpu.VMEM((b_per_w,), jnp.int32),
            pltpu.VMEM((b_per_w, _NA), jnp.float32),
            pltpu.VMEM((_CH, _D), jnp.float32),
            pltpu.VMEM((_CH, _D), jnp.float32),
            pltpu.SemaphoreType.DMA,
            pltpu.SemaphoreType.DMA,
            pltpu.SemaphoreType.DMA,
        ],
    )
    def k(ids_hbm, atoms_hbm, lw_hbm, out_a, out_w,
          idx_v, wrows_v, buf0, buf1, sem0, sem1, semw):
        wid = lax.axis_index("s") * nc + lax.axis_index("c")
        base = wid * b_per_w

        pltpu.sync_copy(ids_hbm.at[pl.ds(base, b_per_w)], idx_v)

        bufs = (buf0, buf1)
        sems = (sem0, sem1)

        cp = pltpu.async_copy(
            atoms_hbm.at[idx_v.at[pl.ds(0, _CH)]], bufs[0], sems[0])
        prev = (cp, 0, 0)

        def wbody(g, carry):
            vec = idx_v[pl.ds(g * 16, 16)]
            for j in range(16):
                t = vec[j]
                pltpu.async_copy(lw_hbm.at[pl.ds(t, 1)],
                                 wrows_v.at[pl.ds(g * 16 + j, 1)], semw)
            return carry
        lax.fori_loop(0, b_per_w // 16, wbody, 0)

        for c in range(1, n_ch):
            s = c & 1
            cp = pltpu.async_copy(
                atoms_hbm.at[idx_v.at[pl.ds(c * _CH, _CH)]], bufs[s], sems[s])
            pcp, ps, pc_i = prev
            pcp.wait()
            pltpu.sync_copy(bufs[ps],
                            out_a.at[pl.ds(base + pc_i * _CH, _CH)])
            prev = (cp, s, c)
        pcp, ps, pc_i = prev
        pcp.wait()
        pltpu.sync_copy(bufs[ps], out_a.at[pl.ds(base + pc_i * _CH, _CH)])

        pltpu.make_async_copy(lw_hbm.at[pl.ds(0, b_per_w)], wrows_v,
                              semw).wait()
        pltpu.sync_copy(wrows_v, out_w.at[pl.ds(base, b_per_w)])

    return k(ids, atom_embeddings, log_weight_embeddings)


def _tc_transpose(a_flat, w_flat, Bt, S):
    B = a_flat.shape[0]
    n_blk = B // _TBLK
    blk_per_row = S // _TBLK

    def body(a_ref, w_ref, oa_ref, ow_ref):
        oa_ref[0] = a_ref[...].T
        ow_ref[0] = w_ref[...].T

    return pl.pallas_call(
        body,
        grid=(n_blk,),
        in_specs=[
            pl.BlockSpec((_TBLK, _D), lambda r: (r, 0)),
            pl.BlockSpec((_TBLK, _NA), lambda r: (r, 0)),
        ],
        out_specs=[
            pl.BlockSpec((1, _D, _TBLK),
                         lambda r: (r // blk_per_row, 0, r % blk_per_row)),
            pl.BlockSpec((1, _NA, _TBLK),
                         lambda r: (r // blk_per_row, 0, r % blk_per_row)),
        ],
        out_shape=(
            jax.ShapeDtypeStruct((Bt, _D, S), jnp.float32),
            jax.ShapeDtypeStruct((Bt, _NA, S), jnp.float32),
        ),
    )(a_flat, w_flat)


def kernel(token_ids, atom_embeddings, log_weight_embeddings):
    Bt, S = token_ids.shape
    ids = token_ids.reshape(-1).astype(jnp.int32)
    a_flat, w_flat = _sc_lookup(ids, atom_embeddings, log_weight_embeddings)
    aT, wT = _tc_transpose(a_flat, w_flat, Bt, S)
    atoms = aT.reshape(Bt, _NA, _DB, S).transpose(0, 3, 1, 2)
    log_weights = wT.transpose(0, 2, 1)
    return (atoms, log_weights)


# speedup vs baseline: 1.8220x; 1.0229x over previous
import functools

import jax
import jax.numpy as jnp
from jax import lax
from jax.experimental import pallas as pl
from jax.experimental.pallas import tpu as pltpu
from jax.experimental.pallas import tpu_sc as plsc

_D = 512
_NA = 8
_DB = 64
_CH = 64
_TBLK = 2048


def _sc_lookup(ids, atom_embeddings, log_weight_embeddings):
    B = ids.shape[0]
    info = plsc.get_sparse_core_info()
    nc, ns = info.num_cores, info.num_subcores
    nw = nc * ns
    b_per_w = B // nw
    n_ch = b_per_w // _CH

    mesh = plsc.VectorSubcoreMesh(core_axis_name="c", subcore_axis_name="s")

    @functools.partial(
        pl.kernel,
        mesh=mesh,
        out_type=(
            jax.ShapeDtypeStruct((B, _D), jnp.float32),
            jax.ShapeDtypeStruct((B, _NA), jnp.float32),
        ),
        scratch_types=[
            pltpu.VMEM((b_per_w,), jnp.int32),
            pltpu.VMEM((b_per_w, _NA), jnp.float32),
            pltpu.VMEM((_CH, _D), jnp.float32),
            pltpu.VMEM((_CH, _D), jnp.float32),
            pltpu.SemaphoreType.DMA,
            pltpu.SemaphoreType.DMA,
            pltpu.SemaphoreType.DMA,
        ],
    )
    def k(ids_hbm, atoms_hbm, lw_hbm, out_a, out_w,
          idx_v, wrows_v, buf0, buf1, sem0, sem1, semw):
        wid = lax.axis_index("s") * nc + lax.axis_index("c")
        base = wid * b_per_w

        pltpu.sync_copy(ids_hbm.at[pl.ds(base, b_per_w)], idx_v)

        bufs = (buf0, buf1)
        sems = (sem0, sem1)

        cp = pltpu.async_copy(
            atoms_hbm.at[idx_v.at[pl.ds(0, _CH)]], bufs[0], sems[0])
        prev = (cp, 0, 0)

        def wbody(g, carry):
            vec = idx_v[pl.ds(g * 16, 16)]
            for j in range(16):
                t = vec[j]
                pltpu.async_copy(lw_hbm.at[pl.ds(t, 1)],
                                 wrows_v.at[pl.ds(g * 16 + j, 1)], semw)
            return carry
        lax.fori_loop(0, b_per_w // 16, wbody, 0)

        for c in range(1, n_ch):
            s = c & 1
            cp = pltpu.async_copy(
                atoms_hbm.at[idx_v.at[pl.ds(c * _CH, _CH)]], bufs[s], sems[s])
            pcp, ps, pc_i = prev
            pcp.wait()
            pltpu.sync_copy(bufs[ps],
                            out_a.at[pl.ds(base + pc_i * _CH, _CH)])
            prev = (cp, s, c)
        pcp, ps, pc_i = prev
        pcp.wait()
        pltpu.sync_copy(bufs[ps], out_a.at[pl.ds(base + pc_i * _CH, _CH)])

        pltpu.make_async_copy(lw_hbm.at[pl.ds(0, b_per_w)], wrows_v,
                              semw).wait()
        pltpu.sync_copy(wrows_v, out_w.at[pl.ds(base, b_per_w)])

    return k(ids, atom_embeddings, log_weight_embeddings)


def _tc_transpose(a_flat, w_flat, Bt, S):
    B = a_flat.shape[0]
    n_blk = B // _TBLK
    blk_per_row = S // _TBLK

    def body(a_ref, w_ref, oa_ref, ow_ref):
        oa_ref[0] = a_ref[...].T
        ow_ref[0] = w_ref[...].T

    return pl.pallas_call(
        body,
        grid=(n_blk,),
        in_specs=[
            pl.BlockSpec((_TBLK, _D), lambda r: (r, 0)),
            pl.BlockSpec((_TBLK, _NA), lambda r: (r, 0)),
        ],
        out_specs=[
            pl.BlockSpec((1, _D, _TBLK),
                         lambda r: (r // blk_per_row, 0, r % blk_per_row)),
            pl.BlockSpec((1, _NA, _TBLK),
                         lambda r: (r // blk_per_row, 0, r % blk_per_row)),
        ],
        out_shape=(
            jax.ShapeDtypeStruct((Bt, _D, S), jnp.float32),
            jax.ShapeDtypeStruct((Bt, _NA, S), jnp.float32),
        ),
    )(a_flat, w_flat)


def kernel(token_ids, atom_embeddings, log_weight_embeddings):
    Bt, S = token_ids.shape
    ids = token_ids.reshape(-1).astype(jnp.int32)
    a_flat, w_flat = _sc_lookup(ids, atom_embeddings, log_weight_embeddings)
    aT, wT = _tc_transpose(a_flat, w_flat, Bt, S)
    atoms = aT.reshape(Bt, _NA, _DB, S).transpose(0, 3, 1, 2)
    log_weights = wT.transpose(0, 2, 1)
    return (atoms, log_weights)


# speedup vs baseline: 1.8471x; 1.0138x over previous
import functools

import jax
import jax.numpy as jnp
from jax import lax
from jax.experimental import pallas as pl
from jax.experimental.pallas import tpu as pltpu
from jax.experimental.pallas import tpu_sc as plsc

_D = 512
_NA = 8
_DB = 64
_CH = 64
_TBLK = 4096


def _sc_lookup(ids, atom_embeddings, log_weight_embeddings, Bt, S):
    B = ids.shape[0]
    info = plsc.get_sparse_core_info()
    nc, ns = info.num_cores, info.num_subcores
    nw = nc * ns
    b_per_w = B // nw
    n_ch = b_per_w // _CH
    w_per_row = S // b_per_w

    mesh = plsc.VectorSubcoreMesh(core_axis_name="c", subcore_axis_name="s")

    @functools.partial(
        pl.kernel,
        mesh=mesh,
        out_type=(
            jax.ShapeDtypeStruct((B, _D), jnp.float32),
            jax.ShapeDtypeStruct((B, _NA), jnp.float32),
        ),
        scratch_types=[
            pltpu.VMEM((b_per_w,), jnp.int32),
            pltpu.VMEM((b_per_w, _NA), jnp.float32),
            pltpu.VMEM((_CH, _D), jnp.float32),
            pltpu.VMEM((_CH, _D), jnp.float32),
            pltpu.SemaphoreType.DMA,
            pltpu.SemaphoreType.DMA,
            pltpu.SemaphoreType.DMA,
        ],
    )
    def k(ids_hbm, atoms_hbm, lw_hbm, out_a, out_w,
          idx_v, wrows_v, buf0, buf1, sem0, sem1, semw):
        wid = lax.axis_index("s") * nc + lax.axis_index("c")
        base = wid * b_per_w
        d0 = wid // w_per_row
        off = (wid % w_per_row) * b_per_w

        pltpu.sync_copy(ids_hbm.at[pl.ds(base, b_per_w)], idx_v)

        bufs = (buf0, buf1)
        sems = (sem0, sem1)

        cp = pltpu.async_copy(
            atoms_hbm.at[idx_v.at[pl.ds(0, _CH)]], bufs[0], sems[0])
        prev = (cp, 0, 0)

        def wbody(g, carry):
            vec = idx_v[pl.ds(g * 16, 16)]
            for j in range(16):
                t = vec[j]
                pltpu.async_copy(lw_hbm.at[pl.ds(t, 1)],
                                 wrows_v.at[pl.ds(g * 16 + j, 1)], semw)
            return carry
        lax.fori_loop(0, b_per_w // 16, wbody, 0)

        for c in range(1, n_ch):
            s = c & 1
            cp = pltpu.async_copy(
                atoms_hbm.at[idx_v.at[pl.ds(c * _CH, _CH)]], bufs[s], sems[s])
            pcp, ps, pc_i = prev
            pcp.wait()
            pltpu.sync_copy(bufs[ps],
                            out_a.at[pl.ds(base + pc_i * _CH, _CH)])
            prev = (cp, s, c)
        pcp, ps, pc_i = prev
        pcp.wait()
        pltpu.sync_copy(bufs[ps], out_a.at[pl.ds(base + pc_i * _CH, _CH)])

        pltpu.make_async_copy(lw_hbm.at[pl.ds(0, b_per_w)], wrows_v,
                              semw).wait()
        pltpu.sync_copy(wrows_v, out_w.at[pl.ds(base, b_per_w)])

    return k(ids, atom_embeddings, log_weight_embeddings)


def _tc_transpose(a_flat, w_flat, Bt, S):
    B = a_flat.shape[0]
    n_blk = B // _TBLK
    d0_per_blk = _TBLK // S

    def body(a_ref, w_ref, oa_ref, ow_ref):
        xt = a_ref[...].T
        wt = w_ref[...].T
        for i in range(d0_per_blk):
            oa_ref[i] = xt[:, i * S:(i + 1) * S]
            ow_ref[i] = wt[:, i * S:(i + 1) * S]

    return pl.pallas_call(
        body,
        grid=(n_blk,),
        in_specs=[
            pl.BlockSpec((_TBLK, _D), lambda r: (r, 0)),
            pl.BlockSpec((_TBLK, _NA), lambda r: (r, 0)),
        ],
        out_specs=[
            pl.BlockSpec((d0_per_blk, _D, S), lambda r: (r, 0, 0)),
            pl.BlockSpec((d0_per_blk, _NA, S), lambda r: (r, 0, 0)),
        ],
        out_shape=(
            jax.ShapeDtypeStruct((Bt, _D, S), jnp.float32),
            jax.ShapeDtypeStruct((Bt, _NA, S), jnp.float32),
        ),
    )(a_flat, w_flat)


def kernel(token_ids, atom_embeddings, log_weight_embeddings):
    Bt, S = token_ids.shape
    ids = token_ids.reshape(-1).astype(jnp.int32)
    a_flat, w_flat = _sc_lookup(ids, atom_embeddings, log_weight_embeddings,
                                Bt, S)
    aT, wT = _tc_transpose(a_flat, w_flat, Bt, S)
    atoms = aT.reshape(Bt, _NA, _DB, S).transpose(0, 3, 1, 2)
    log_weights = wT.transpose(0, 2, 1)
    return (atoms, log_weights)


# speedup vs baseline: 2.0689x; 1.1201x over previous
import functools

import jax
import jax.numpy as jnp
from jax import lax
from jax.experimental import pallas as pl
from jax.experimental.pallas import tpu as pltpu
from jax.experimental.pallas import tpu_sc as plsc

_D = 512
_NA = 8
_DB = 64
_CH = 64
_WV = 16
_TBLK = 4096


def _sc_lookup(ids, atom_embeddings, lwT, Bt, S):
    B = ids.shape[0]
    info = plsc.get_sparse_core_info()
    nc, ns = info.num_cores, info.num_subcores
    nw = nc * ns
    b_per_w = B // nw
    n_ch = b_per_w // _CH
    n_wv = b_per_w // _WV
    w_per_row = S // b_per_w

    mesh = plsc.VectorSubcoreMesh(core_axis_name="c", subcore_axis_name="s")

    @functools.partial(
        pl.kernel,
        mesh=mesh,
        out_type=(
            jax.ShapeDtypeStruct((B, _D), jnp.float32),
            jax.ShapeDtypeStruct((Bt, _NA, S), jnp.float32),
        ),
        scratch_types=[
            pltpu.VMEM((b_per_w,), jnp.int32),
            pltpu.VMEM((_WV * _NA, 128), jnp.float32),
            pltpu.VMEM((_WV * _NA, 128), jnp.float32),
            pltpu.VMEM((_NA, b_per_w), jnp.float32),
            pltpu.VMEM((_CH, _D), jnp.float32),
            pltpu.VMEM((_CH, _D), jnp.float32),
            pltpu.SemaphoreType.DMA,
            pltpu.SemaphoreType.DMA,
            pltpu.SemaphoreType.DMA,
            pltpu.SemaphoreType.DMA,
        ],
    )
    def k(ids_hbm, atoms_hbm, lw_hbm, out_a, out_w,
          idx_v, wgA, wgB, wrowsT, buf0, buf1, sem0, sem1, semwA, semwB):
        wid = lax.axis_index("s") * nc + lax.axis_index("c")
        base = wid * b_per_w
        d0 = wid // w_per_row
        off = (wid % w_per_row) * b_per_w

        pltpu.sync_copy(ids_hbm.at[pl.ds(base, b_per_w)], idx_v)

        bufs = (buf0, buf1)
        sems = (sem0, sem1)
        wgs = (wgA, wgB)
        wsems = (semwA, semwB)
        iot = lax.iota(jnp.int32, 16)
        zf = jnp.zeros((16,), jnp.float32)

        def agather(c):
            return pltpu.async_copy(
                atoms_hbm.at[idx_v.at[pl.ds(c * _CH, _CH)]],
                bufs[c & 1], sems[c & 1])

        def wfetch(w):
            wg, ws = wgs[w & 1], wsems[w & 1]
            tvec = idx_v[pl.ds(w * _WV, _WV)]
            for j in range(_WV):
                t = tvec[j]
                g0 = (t >> 7) * 128
                pltpu.async_copy(lw_hbm.at[pl.ds(0, _NA), pl.ds(g0, 128)],
                                 wg.at[pl.ds(j * _NA, _NA)], ws)
            return tvec

        def wextract(w, tvec):
            wg, ws = wgs[w & 1], wsems[w & 1]
            pltpu.make_async_copy(
                atoms_hbm.at[pl.ds(0, _WV * _NA), pl.ds(0, 128)], wg,
                ws).wait()

            def jb(j, carry):
                tv = carry[0]
                accs = list(carry[1:])
                t = tv[0]
                w16 = ((t & 127) >> 4) << 4
                l15 = iot * 0 + (t & 15)
                mask = iot == j
                for a in range(_NA):
                    va = wg[j * _NA + a, pl.ds(w16, 16)]
                    dgv = jnp.take(va, l15)
                    accs[a] = jnp.where(mask, dgv, accs[a])
                tv2 = jnp.take(tv, (iot + 1) & 15)
                return (tv2, *accs)

            carry = lax.fori_loop(0, _WV, jb, (tvec, *([zf] * _NA)))
            for a in range(_NA):
                wrowsT[a, pl.ds(w * _WV, _WV)] = carry[1 + a]

        gcp = {0: agather(0), 1: agather(1)}
        tvs = {0: wfetch(0)}
        for c in range(n_ch):
            for k_ in range(n_wv // n_ch):
                w = c * (n_wv // n_ch) + k_
                if w + 1 < n_wv:
                    tvs[w + 1] = wfetch(w + 1)
                wextract(w, tvs.pop(w))
            gcp[c].wait()
            pltpu.sync_copy(bufs[c & 1], out_a.at[pl.ds(base + c * _CH, _CH)])
            if c + 2 < n_ch:
                gcp[c + 2] = agather(c + 2)
        pltpu.sync_copy(wrowsT, out_w.at[d0, :, pl.ds(off, b_per_w)])

    return k(ids, atom_embeddings, lwT)


def _tc_transpose(a_flat, Bt, S):
    B = a_flat.shape[0]
    n_blk = B // _TBLK
    d0_per_blk = _TBLK // S

    def body(a_ref, oa_ref):
        xt = a_ref[...].T
        for i in range(d0_per_blk):
            oa_ref[i] = xt[:, i * S:(i + 1) * S]

    return pl.pallas_call(
        body,
        grid=(n_blk,),
        in_specs=[pl.BlockSpec((_TBLK, _D), lambda r: (r, 0))],
        out_specs=pl.BlockSpec((d0_per_blk, _D, S), lambda r: (r, 0, 0)),
        out_shape=jax.ShapeDtypeStruct((Bt, _D, S), jnp.float32),
    )(a_flat)


def kernel(token_ids, atom_embeddings, log_weight_embeddings):
    Bt, S = token_ids.shape
    ids = token_ids.reshape(-1).astype(jnp.int32)
    a_flat, wT = _sc_lookup(ids, atom_embeddings,
                            log_weight_embeddings.T, Bt, S)
    aT = _tc_transpose(a_flat, Bt, S)
    atoms = aT.reshape(Bt, _NA, _DB, S).transpose(0, 3, 1, 2)
    log_weights = wT.transpose(0, 2, 1)
    return (atoms, log_weights)
